# baseline (speedup 1.0000x reference)
import math

import jax
import jax.numpy as jnp
from jax.experimental import pallas as pl
from jax.experimental.pallas import tpu as pltpu

D_MODEL = 384
D_HEAD = 64
N_HEAD = D_MODEL // D_HEAD
D_MID = D_MODEL * 4
B, H, W = 4, 32, 32
N_TOK = H * W
EPS = 1e-5
PAD = 40
MID_CHUNK = 512
N_CHUNK = D_MID // MID_CHUNK

_F32 = jnp.float32
_BF16 = jnp.bfloat16


def _gelu(x):
    return 0.5 * x * (1.0 + jax.lax.erf(x * (1.0 / math.sqrt(2.0))))


def _gelu2(x):
    return x * (1.0 + jax.lax.erf(x * (1.0 / math.sqrt(2.0))))


def _block_body(x0_ref, qw_ref, kw_ref, vw_ref, mw_ref, vec_ref,
                w1_ref, w2_ref, dwt_ref, bmid_ref, b3_ref, out_ref):
    x0 = x0_ref[0]
    sc0 = vec_ref[0:1, :]
    b0 = vec_ref[1:2, :]
    mb = vec_ref[2:3, :]
    xg = _gelu2(x0 * sc0 + b0).astype(_BF16)
    ones = jnp.ones((N_TOK, D_HEAD), _BF16)
    dn_cc = (((1,), (1,)), ((), ()))
    qa = jax.lax.dot_general(xg, qw_ref[...], dn_cc,
                             preferred_element_type=_F32).astype(_BF16)
    ka = jax.lax.dot_general(xg, kw_ref[...], dn_cc,
                             preferred_element_type=_F32).astype(_BF16)
    va = jax.lax.dot_general(xg, vw_ref[...], dn_cc,
                             preferred_element_type=_F32).astype(_BF16)
    mhs = []
    for h in range(N_HEAD):
        qh = jax.lax.slice(qa, (0, h * D_HEAD), (N_TOK, (h + 1) * D_HEAD))
        kh = jax.lax.slice(ka, (0, h * D_HEAD), (N_TOK, (h + 1) * D_HEAD))
        vh = jax.lax.slice(va, (0, h * D_HEAD), (N_TOK, (h + 1) * D_HEAD))
        s = jax.lax.dot_general(qh, kh, dn_cc, preferred_element_type=_F32)
        p = jnp.exp(jnp.minimum(s, 40.0)).astype(_BF16)
        vext = jnp.concatenate([vh, ones], axis=1)
        mv = jax.lax.dot_general(p, vext, (((1,), (0,)), ((), ())),
                                 preferred_element_type=_F32)
        l = jax.lax.slice(mv, (0, D_HEAD), (N_TOK, D_HEAD + 1))
        mhs.append((jax.lax.slice(mv, (0, 0), (N_TOK, D_HEAD)) / l).astype(_BF16))
    msg = jnp.concatenate(mhs, axis=1)
    x = x0 + mb + jax.lax.dot_general(
        msg, mw_ref[...], dn_cc, preferred_element_type=_F32)
    xb = x.astype(_BF16)
    col = jax.lax.broadcasted_iota(jnp.int32, (N_TOK, 1), 0) % W
    m_l = col >= 1
    m_r = col <= W - 2
    z8 = jnp.zeros((8, MID_CHUNK), _F32)
    zW = jnp.zeros((W, MID_CHUNK), _F32)
    acc = x + b3_ref[0:1, :]
    for nc in range(N_CHUNK):
        c0 = nc * MID_CHUNK
        y = _gelu2(jax.lax.dot_general(xb, w1_ref[c0:c0 + MID_CHUNK, :],
                                       dn_cc, preferred_element_type=_F32)
                   + bmid_ref[0:1, c0:c0 + MID_CHUNK])
        yp8 = jnp.concatenate([z8, y, z8], axis=0)
        um = jnp.where(m_l,
                       jax.lax.slice(yp8, (7, 0), (7 + N_TOK, MID_CHUNK)), 0.0)
        up = jnp.where(m_r,
                       jax.lax.slice(yp8, (9, 0), (9 + N_TOK, MID_CHUNK)), 0.0)

        def rowconv(i, um=um, y=y, up=up, c0=c0):
            return (um * dwt_ref[3 * i:3 * i + 1, c0:c0 + MID_CHUNK]
                    + y * dwt_ref[3 * i + 1:3 * i + 2, c0:c0 + MID_CHUNK]
                    + up * dwt_ref[3 * i + 2:3 * i + 3, c0:c0 + MID_CHUNK])

        cm1p = jnp.concatenate([zW, rowconv(0), zW], axis=0)
        cp1p = jnp.concatenate([zW, rowconv(2), zW], axis=0)
        z = (rowconv(1)
             + jax.lax.slice(cm1p, (0, 0), (N_TOK, MID_CHUNK))
             + jax.lax.slice(cp1p, (2 * W, 0), (2 * W + N_TOK, MID_CHUNK)))
        z = _gelu2(z + bmid_ref[1:2, c0:c0 + MID_CHUNK])
        acc = acc + jax.lax.dot_general(
            z.astype(_BF16), w2_ref[:, c0:c0 + MID_CHUNK], dn_cc,
            preferred_element_type=_F32)
    out_ref[0] = acc


@jax.jit
def kernel(x0, bn0_g, bn0_b, q_w, k_w, v_w, merge_w, merge_b,
           mlp_w1, mlp_bn1_g, mlp_bn1_b, mlp_dw, mlp_bn2_g, mlp_bn2_b,
           mlp_w2, mlp_bn3_g, mlp_bn3_b):
    inv = 1.0 / math.sqrt(1.0 + EPS)
    x0t = x0.reshape(B, D_MODEL, N_TOK).transpose(0, 2, 1)

    qw2 = (q_w * (0.5 / math.sqrt(D_HEAD))).astype(_BF16)
    kw2 = (k_w * 0.5).astype(_BF16)
    vw2 = (v_w * 0.5).astype(_BF16)
    mw2 = merge_w.astype(_BF16)

    vec1 = jnp.zeros((8, D_MODEL), _F32)
    vec1 = vec1.at[0].set(bn0_g * inv).at[1].set(bn0_b).at[2].set(merge_b)

    w1f = (mlp_w1 * (mlp_bn1_g * inv)[:, None]).astype(_BF16)
    w2f = (mlp_w2 * (0.5 * mlp_bn3_g * inv)[:, None]).astype(_BF16)
    dwt = jnp.zeros((16, D_MID), _F32)
    dwt = dwt.at[:9].set((mlp_dw.reshape(D_MID, 9)
                          * (0.5 * mlp_bn2_g * inv)[:, None]).T)
    bmid = jnp.zeros((8, D_MID), _F32)
    bmid = bmid.at[0].set(mlp_bn1_b).at[1].set(mlp_bn2_b)
    b3 = jnp.zeros((8, D_MODEL), _F32)
    b3 = b3.at[0].set(mlp_bn3_b)

    out = pl.pallas_call(
        _block_body,
        grid=(B,),
        in_specs=[
            pl.BlockSpec((1, N_TOK, D_MODEL), lambda b: (b, 0, 0)),
            pl.BlockSpec((D_MODEL, D_MODEL), lambda b: (0, 0)),
            pl.BlockSpec((D_MODEL, D_MODEL), lambda b: (0, 0)),
            pl.BlockSpec((D_MODEL, D_MODEL), lambda b: (0, 0)),
            pl.BlockSpec((D_MODEL, D_MODEL), lambda b: (0, 0)),
            pl.BlockSpec((8, D_MODEL), lambda b: (0, 0)),
            pl.BlockSpec((D_MID, D_MODEL), lambda b: (0, 0)),
            pl.BlockSpec((D_MODEL, D_MID), lambda b: (0, 0)),
            pl.BlockSpec((16, D_MID), lambda b: (0, 0)),
            pl.BlockSpec((8, D_MID), lambda b: (0, 0)),
            pl.BlockSpec((8, D_MODEL), lambda b: (0, 0)),
        ],
        out_specs=pl.BlockSpec((1, N_TOK, D_MODEL), lambda b: (b, 0, 0)),
        out_shape=jax.ShapeDtypeStruct((B, N_TOK, D_MODEL), _F32),
        compiler_params=pltpu.CompilerParams(
            dimension_semantics=("parallel",)),
    )(x0t, qw2, kw2, vw2, mw2, vec1, w1f, w2f, dwt, bmid, b3)

    return out.transpose(0, 2, 1).reshape(B, D_MODEL, H, W)


# speedup vs baseline: 1.0209x; 1.0209x over previous
import math

import jax
import jax.numpy as jnp
from jax.experimental import pallas as pl
from jax.experimental.pallas import tpu as pltpu

D_MODEL = 384
D_HEAD = 64
N_HEAD = D_MODEL // D_HEAD
D_MID = D_MODEL * 4
B, H, W = 4, 32, 32
N_TOK = H * W
EPS = 1e-5
MID_CHUNK = 768
N_CHUNK = D_MID // MID_CHUNK

_F32 = jnp.float32
_BF16 = jnp.bfloat16


def _gelu2(x):
    return x * (1.0 + jax.lax.erf(x * (1.0 / math.sqrt(2.0))))


def _block_body(x0_ref, qw_ref, kw_ref, vw_ref, mw_ref, vec_ref,
                w1_ref, w2_ref, dwt_ref, bmid_ref, b3_ref, out_ref):
    x0 = x0_ref[0]
    sc0 = vec_ref[0:1, :]
    b0 = vec_ref[1:2, :]
    mb = vec_ref[2:3, :]
    xg = _gelu2(x0 * sc0 + b0).astype(_BF16)
    ones = jnp.ones((N_TOK, D_HEAD), _BF16)
    dn_cc = (((1,), (1,)), ((), ()))
    qa = jax.lax.dot_general(xg, qw_ref[...], dn_cc,
                             preferred_element_type=_F32).astype(_BF16)
    ka = jax.lax.dot_general(xg, kw_ref[...], dn_cc,
                             preferred_element_type=_F32).astype(_BF16)
    va = jax.lax.dot_general(xg, vw_ref[...], dn_cc,
                             preferred_element_type=_F32).astype(_BF16)
    mhs = []
    for h in range(N_HEAD):
        qh = jax.lax.slice(qa, (0, h * D_HEAD), (N_TOK, (h + 1) * D_HEAD))
        kh = jax.lax.slice(ka, (0, h * D_HEAD), (N_TOK, (h + 1) * D_HEAD))
        vh = jax.lax.slice(va, (0, h * D_HEAD), (N_TOK, (h + 1) * D_HEAD))
        s = jax.lax.dot_general(qh, kh, dn_cc, preferred_element_type=_F32)
        p = jnp.exp(jnp.minimum(s, 40.0)).astype(_BF16)
        vext = jnp.concatenate([vh, ones], axis=1)
        mv = jax.lax.dot_general(p, vext, (((1,), (0,)), ((), ())),
                                 preferred_element_type=_F32)
        l = jax.lax.slice(mv, (0, D_HEAD), (N_TOK, D_HEAD + 1))
        mhs.append((jax.lax.slice(mv, (0, 0), (N_TOK, D_HEAD)) / l).astype(_BF16))
    msg = jnp.concatenate(mhs, axis=1)
    x = x0 + mb + jax.lax.dot_general(
        msg, mw_ref[...], dn_cc, preferred_element_type=_F32)
    xb = x.astype(_BF16)
    col = jax.lax.broadcasted_iota(jnp.int32, (N_TOK, 1), 0) % W
    m_l = col >= 1
    m_r = col <= W - 2
    z8 = jnp.zeros((8, MID_CHUNK), _F32)
    zW = jnp.zeros((W, MID_CHUNK), _F32)
    acc = x + b3_ref[0:1, :]
    for nc in range(N_CHUNK):
        c0 = nc * MID_CHUNK
        y = _gelu2(jax.lax.dot_general(xb, w1_ref[c0:c0 + MID_CHUNK, :],
                                       dn_cc, preferred_element_type=_F32)
                   + bmid_ref[0:1, c0:c0 + MID_CHUNK])
        yp8 = jnp.concatenate([z8, y, z8], axis=0)
        um = jnp.where(m_l,
                       jax.lax.slice(yp8, (7, 0), (7 + N_TOK, MID_CHUNK)), 0.0)
        up = jnp.where(m_r,
                       jax.lax.slice(yp8, (9, 0), (9 + N_TOK, MID_CHUNK)), 0.0)

        def rowconv(i, um=um, y=y, up=up, c0=c0):
            return (um * dwt_ref[3 * i:3 * i + 1, c0:c0 + MID_CHUNK]
                    + y * dwt_ref[3 * i + 1:3 * i + 2, c0:c0 + MID_CHUNK]
                    + up * dwt_ref[3 * i + 2:3 * i + 3, c0:c0 + MID_CHUNK])

        cm1p = jnp.concatenate([zW, rowconv(0), zW], axis=0)
        cp1p = jnp.concatenate([zW, rowconv(2), zW], axis=0)
        z = (rowconv(1)
             + jax.lax.slice(cm1p, (0, 0), (N_TOK, MID_CHUNK))
             + jax.lax.slice(cp1p, (2 * W, 0), (2 * W + N_TOK, MID_CHUNK)))
        z = _gelu2(z + bmid_ref[1:2, c0:c0 + MID_CHUNK])
        acc = acc + jax.lax.dot_general(
            z.astype(_BF16), w2_ref[:, c0:c0 + MID_CHUNK], dn_cc,
            preferred_element_type=_F32)
    out_ref[0] = acc


@jax.jit
def kernel(x0, bn0_g, bn0_b, q_w, k_w, v_w, merge_w, merge_b,
           mlp_w1, mlp_bn1_g, mlp_bn1_b, mlp_dw, mlp_bn2_g, mlp_bn2_b,
           mlp_w2, mlp_bn3_g, mlp_bn3_b):
    inv = 1.0 / math.sqrt(1.0 + EPS)
    x0t = x0.reshape(B, D_MODEL, N_TOK).transpose(0, 2, 1)

    qw2 = (q_w * (0.5 / math.sqrt(D_HEAD))).astype(_BF16)
    kw2 = (k_w * 0.5).astype(_BF16)
    vw2 = (v_w * 0.5).astype(_BF16)
    mw2 = merge_w.astype(_BF16)

    vec1 = jnp.zeros((8, D_MODEL), _F32)
    vec1 = vec1.at[0].set(bn0_g * inv).at[1].set(bn0_b).at[2].set(merge_b)

    w1f = (mlp_w1 * (mlp_bn1_g * inv)[:, None]).astype(_BF16)
    w2f = (mlp_w2 * (0.5 * mlp_bn3_g * inv)[:, None]).astype(_BF16)
    dwt = jnp.zeros((16, D_MID), _F32)
    dwt = dwt.at[:9].set((mlp_dw.reshape(D_MID, 9)
                          * (0.5 * mlp_bn2_g * inv)[:, None]).T)
    bmid = jnp.zeros((8, D_MID), _F32)
    bmid = bmid.at[0].set(mlp_bn1_b).at[1].set(mlp_bn2_b)
    b3 = jnp.zeros((8, D_MODEL), _F32)
    b3 = b3.at[0].set(mlp_bn3_b)

    out = pl.pallas_call(
        _block_body,
        grid=(B,),
        in_specs=[
            pl.BlockSpec((1, N_TOK, D_MODEL), lambda b: (b, 0, 0)),
            pl.BlockSpec((D_MODEL, D_MODEL), lambda b: (0, 0)),
            pl.BlockSpec((D_MODEL, D_MODEL), lambda b: (0, 0)),
            pl.BlockSpec((D_MODEL, D_MODEL), lambda b: (0, 0)),
            pl.BlockSpec((D_MODEL, D_MODEL), lambda b: (0, 0)),
            pl.BlockSpec((8, D_MODEL), lambda b: (0, 0)),
            pl.BlockSpec((D_MID, D_MODEL), lambda b: (0, 0)),
            pl.BlockSpec((D_MODEL, D_MID), lambda b: (0, 0)),
            pl.BlockSpec((16, D_MID), lambda b: (0, 0)),
            pl.BlockSpec((8, D_MID), lambda b: (0, 0)),
            pl.BlockSpec((8, D_MODEL), lambda b: (0, 0)),
        ],
        out_specs=pl.BlockSpec((1, N_TOK, D_MODEL), lambda b: (b, 0, 0)),
        out_shape=jax.ShapeDtypeStruct((B, N_TOK, D_MODEL), _F32),
        compiler_params=pltpu.CompilerParams(
            dimension_semantics=("parallel",)),
    )(x0t, qw2, kw2, vw2, mw2, vec1, w1f, w2f, dwt, bmid, b3)

    return out.transpose(0, 2, 1).reshape(B, D_MODEL, H, W)


# speedup vs baseline: 1.0259x; 1.0049x over previous
import math

import jax
import jax.numpy as jnp
from jax.experimental import pallas as pl
from jax.experimental.pallas import tpu as pltpu

D_MODEL = 384
D_HEAD = 64
N_HEAD = D_MODEL // D_HEAD
D_MID = D_MODEL * 4
B, H, W = 4, 32, 32
N_TOK = H * W
EPS = 1e-5
MID_CHUNK = 768
N_CHUNK = D_MID // MID_CHUNK

_F32 = jnp.float32
_BF16 = jnp.bfloat16


def _gelu2(x):
    return x * (1.0 + jax.lax.erf(x * (1.0 / math.sqrt(2.0))))


def _block_body(x0_ref, qw_ref, kw_ref, vw_ref, mw_ref, vec_ref,
                w1_ref, w2_ref, dwt_ref, bmid_ref, b3_ref, out_ref):
    x0 = x0_ref[0]
    sc0 = vec_ref[0:1, :]
    b0 = vec_ref[1:2, :]
    mb = vec_ref[2:3, :]
    xg = _gelu2(x0 * sc0 + b0).astype(_BF16)
    ones = jnp.ones((N_TOK, D_HEAD), _BF16)
    dn_cc = (((1,), (1,)), ((), ()))
    qa = jax.lax.dot_general(xg, qw_ref[...], dn_cc,
                             preferred_element_type=_F32).astype(_BF16)
    ka = jax.lax.dot_general(xg, kw_ref[...], dn_cc,
                             preferred_element_type=_F32).astype(_BF16)
    va = jax.lax.dot_general(xg, vw_ref[...], dn_cc,
                             preferred_element_type=_F32).astype(_BF16)
    mhs = []
    for h in range(N_HEAD):
        qh = jax.lax.slice(qa, (0, h * D_HEAD), (N_TOK, (h + 1) * D_HEAD))
        kh = jax.lax.slice(ka, (0, h * D_HEAD), (N_TOK, (h + 1) * D_HEAD))
        vh = jax.lax.slice(va, (0, h * D_HEAD), (N_TOK, (h + 1) * D_HEAD))
        s = jax.lax.dot_general(qh, kh, dn_cc, preferred_element_type=_F32)
        p = jnp.exp(jnp.minimum(s, 40.0)).astype(_BF16)
        vext = jnp.concatenate([vh, ones], axis=1)
        mv = jax.lax.dot_general(p, vext, (((1,), (0,)), ((), ())),
                                 preferred_element_type=_F32)
        l = jax.lax.slice(mv, (0, D_HEAD), (N_TOK, D_HEAD + 1))
        mhs.append((jax.lax.slice(mv, (0, 0), (N_TOK, D_HEAD)) / l).astype(_BF16))
    msg = jnp.concatenate(mhs, axis=1)
    x = x0 + mb + jax.lax.dot_general(
        msg, mw_ref[...], dn_cc, preferred_element_type=_F32)
    xb = x.astype(_BF16)
    col = jax.lax.broadcasted_iota(jnp.int32, (N_TOK, 1), 0) % W
    m_l = col >= 1
    m_r = col <= W - 2
    z8 = jnp.zeros((8, MID_CHUNK), _F32)
    zW = jnp.zeros((W, MID_CHUNK), _F32)
    acc = x + b3_ref[0:1, :]
    for nc in range(N_CHUNK):
        c0 = nc * MID_CHUNK
        y = _gelu2(jax.lax.dot_general(xb, w1_ref[c0:c0 + MID_CHUNK, :],
                                       dn_cc, preferred_element_type=_F32)
                   + bmid_ref[0:1, c0:c0 + MID_CHUNK])
        yp8 = jnp.concatenate([z8, y, z8], axis=0)
        um = jnp.where(m_l,
                       jax.lax.slice(yp8, (7, 0), (7 + N_TOK, MID_CHUNK)), 0.0)
        up = jnp.where(m_r,
                       jax.lax.slice(yp8, (9, 0), (9 + N_TOK, MID_CHUNK)), 0.0)

        def rowconv(i, um=um, y=y, up=up, c0=c0):
            return (um * dwt_ref[3 * i:3 * i + 1, c0:c0 + MID_CHUNK]
                    + y * dwt_ref[3 * i + 1:3 * i + 2, c0:c0 + MID_CHUNK]
                    + up * dwt_ref[3 * i + 2:3 * i + 3, c0:c0 + MID_CHUNK])

        cm1p = jnp.concatenate([zW, rowconv(0), zW], axis=0)
        cp1p = jnp.concatenate([zW, rowconv(2), zW], axis=0)
        z = (rowconv(1)
             + jax.lax.slice(cm1p, (0, 0), (N_TOK, MID_CHUNK))
             + jax.lax.slice(cp1p, (2 * W, 0), (2 * W + N_TOK, MID_CHUNK)))
        z = _gelu2(z + bmid_ref[1:2, c0:c0 + MID_CHUNK])
        acc = acc + jax.lax.dot_general(
            z.astype(_BF16), w2_ref[:, c0:c0 + MID_CHUNK], dn_cc,
            preferred_element_type=_F32)
    out_ref[0] = acc


@jax.jit
def kernel(x0, bn0_g, bn0_b, q_w, k_w, v_w, merge_w, merge_b,
           mlp_w1, mlp_bn1_g, mlp_bn1_b, mlp_dw, mlp_bn2_g, mlp_bn2_b,
           mlp_w2, mlp_bn3_g, mlp_bn3_b):
    inv = 1.0 / math.sqrt(1.0 + EPS)
    x0t = x0.reshape(B, D_MODEL, N_TOK).transpose(0, 2, 1)

    qw2 = (q_w * (0.5 / math.sqrt(D_HEAD))).astype(_BF16)
    kw2 = (k_w * 0.5).astype(_BF16)
    vw2 = (v_w * 0.5).astype(_BF16)
    mw2 = merge_w.astype(_BF16)

    vec1 = jnp.zeros((8, D_MODEL), _F32)
    vec1 = vec1.at[0].set(bn0_g * inv).at[1].set(bn0_b).at[2].set(merge_b)

    w1f = (mlp_w1 * (mlp_bn1_g * inv)[:, None]).astype(_BF16)
    w2f = (mlp_w2 * (0.5 * mlp_bn3_g * inv)[:, None]).astype(_BF16)
    dwt = jnp.zeros((16, D_MID), _F32)
    dwt = dwt.at[:9].set((mlp_dw.reshape(D_MID, 9)
                          * (0.5 * mlp_bn2_g * inv)[:, None]).T)
    bmid = jnp.zeros((8, D_MID), _F32)
    bmid = bmid.at[0].set(mlp_bn1_b).at[1].set(mlp_bn2_b)
    b3 = jnp.zeros((8, D_MODEL), _F32)
    b3 = b3.at[0].set(mlp_bn3_b)

    out = pl.pallas_call(
        _block_body,
        grid=(B,),
        in_specs=[
            pl.BlockSpec((1, N_TOK, D_MODEL), lambda b: (b, 0, 0)),
            pl.BlockSpec((D_MODEL, D_MODEL), lambda b: (0, 0)),
            pl.BlockSpec((D_MODEL, D_MODEL), lambda b: (0, 0)),
            pl.BlockSpec((D_MODEL, D_MODEL), lambda b: (0, 0)),
            pl.BlockSpec((D_MODEL, D_MODEL), lambda b: (0, 0)),
            pl.BlockSpec((8, D_MODEL), lambda b: (0, 0)),
            pl.BlockSpec((D_MID, D_MODEL), lambda b: (0, 0)),
            pl.BlockSpec((D_MODEL, D_MID), lambda b: (0, 0)),
            pl.BlockSpec((16, D_MID), lambda b: (0, 0)),
            pl.BlockSpec((8, D_MID), lambda b: (0, 0)),
            pl.BlockSpec((8, D_MODEL), lambda b: (0, 0)),
        ],
        out_specs=pl.BlockSpec((1, N_TOK, D_MODEL), lambda b: (b, 0, 0)),
        out_shape=jax.ShapeDtypeStruct((B, N_TOK, D_MODEL), _F32),
        compiler_params=pltpu.CompilerParams(
            dimension_semantics=("arbitrary",)),
    )(x0t, qw2, kw2, vw2, mw2, vec1, w1f, w2f, dwt, bmid, b3)

    return out.transpose(0, 2, 1).reshape(B, D_MODEL, H, W)
